# baseline (speedup 1.0000x reference)
import functools

import jax
import jax.numpy as jnp
from jax import lax
from jax.experimental import pallas as pl
from jax.experimental.pallas import tpu as pltpu
from jax.experimental.pallas import tpu_sc as plsc

NUM_CARDS = 1000000
HIDDEN_DIM = 32
BATCH = 4096
HIST = 200

_info = plsc.get_sparse_core_info()
NC, NS, L = _info.num_cores, _info.num_subcores, _info.num_lanes
NW = NC * NS

NB = BATCH // NW
B_PER_W = NB * HIST
TCH = 4
CROWS = TCH * NB
NCH = HIST // TCH
NPAIR = NCH // 2
DG = HIDDEN_DIM // 8


def _body(cards_hbm, emb_hbm, out_hbm, idx_all, idxT, rows0, rows1,
          tb0, tb1, tb2, tb3, sg0, sg1, sw0, sw1, sw2, sw3):
    wid = lax.axis_index("s") * NC + lax.axis_index("c")
    base = wid * B_PER_W
    iota = lax.iota(jnp.int32, L)

    pltpu.sync_copy(cards_hbm.at[pl.ds(base, B_PER_W)], idx_all)

    def build(i, _):
        t = i // (NB // L)
        bg = i % (NB // L)
        src = (bg * L + iota) * HIST + t
        vals = plsc.load_gather(idx_all, [src]) + 1
        idxT[pl.ds(t * NB + bg * L, L)] = vals
        return 0

    lax.fori_loop(0, HIST * (NB // L), build, 0, unroll=8)

    def idx_slice(c):
        return idxT.at[pl.ds(pl.multiple_of(c * CROWS, CROWS), CROWS)]

    def start_gather(c, rows, sem):
        pltpu.async_copy(emb_hbm.at[idx_slice(c)], rows, sem)

    def wait_gather(c, rows, sem):
        pltpu.make_async_copy(emb_hbm.at[idx_slice(c)], rows, sem).wait()

    def start_write(t, tb, sem):
        for g in range(DG):
            pltpu.async_copy(tb.at[pl.ds(g * 8, 8)], out_hbm.at[t, g, wid], sem)

    def wait_write(t, tb, sem):
        for g in range(DG):
            pltpu.make_async_copy(
                tb.at[pl.ds(g * 8, 8)], out_hbm.at[t, g, wid], sem
            ).wait()

    tbufs = (tb0, tb1, tb2, tb3)
    swaps = (sw0, sw1, sw2, sw3)

    def transpose_t(rows, tl, tb):
        rbase = tl * NB
        for bg in range(NB // L):
            rid = rbase + bg * L + iota

            def col(d, _):
                dvec = jnp.broadcast_to(d, (L,))
                vals = plsc.load_gather(rows, [rid, dvec])
                tb[d, pl.ds(bg * L, L)] = vals
                return 0

            lax.fori_loop(0, HIDDEN_DIM, col, 0, unroll=4)

    def handle_chunk(c, rows, rows_nxt, sem, sem_nxt):
        @pl.when(c + 1 < NCH)
        def _():
            start_gather(c + 1, rows_nxt, sem_nxt)

        wait_gather(c, rows, sem)
        for tl in range(TCH):
            t = c * TCH + tl

            @pl.when(c >= 1)
            def _():
                wait_write(t - TCH, tbufs[tl], swaps[tl])

            transpose_t(rows, tl, tbufs[tl])
            start_write(t, tbufs[tl], swaps[tl])

    start_gather(0, rows0, sg0)

    def pair(g, _):
        c0 = 2 * g
        handle_chunk(c0, rows0, rows1, sg0, sg1)
        handle_chunk(c0 + 1, rows1, rows0, sg1, sg0)
        return 0

    lax.fori_loop(0, NPAIR, pair, 0)

    for tl in range(TCH):
        t = HIST - TCH + tl
        wait_write(t, tbufs[tl], (sw0, sw1, sw2, sw3)[tl])


@jax.jit
def _embed(cards_flat, emb):
    mesh = plsc.VectorSubcoreMesh(core_axis_name="c", subcore_axis_name="s")
    fn = pl.kernel(
        _body,
        out_type=jax.ShapeDtypeStruct((HIST, DG, NW, 8, NB), jnp.float32),
        mesh=mesh,
        scratch_types=[
            pltpu.VMEM((B_PER_W,), jnp.int32),
            pltpu.VMEM((B_PER_W,), jnp.int32),
            pltpu.VMEM((CROWS, HIDDEN_DIM), jnp.float32),
            pltpu.VMEM((CROWS, HIDDEN_DIM), jnp.float32),
            pltpu.VMEM((HIDDEN_DIM, NB), jnp.float32),
            pltpu.VMEM((HIDDEN_DIM, NB), jnp.float32),
            pltpu.VMEM((HIDDEN_DIM, NB), jnp.float32),
            pltpu.VMEM((HIDDEN_DIM, NB), jnp.float32),
            pltpu.SemaphoreType.DMA,
            pltpu.SemaphoreType.DMA,
            pltpu.SemaphoreType.DMA,
            pltpu.SemaphoreType.DMA,
            pltpu.SemaphoreType.DMA,
            pltpu.SemaphoreType.DMA,
        ],
        compiler_params=pltpu.CompilerParams(
            use_tc_tiling_on_sc=False, needs_layout_passes=False
        ),
    )
    return fn(cards_flat, emb)


def kernel(cards, emb):
    cards_flat = cards.reshape(-1).astype(jnp.int32)
    out5 = _embed(cards_flat, emb)
    return out5.transpose(2, 4, 0, 1, 3).reshape(BATCH, HIST, HIDDEN_DIM)


# speedup vs baseline: 1.1398x; 1.1398x over previous
import functools

import jax
import jax.numpy as jnp
from jax import lax
from jax.experimental import pallas as pl
from jax.experimental.pallas import tpu as pltpu
from jax.experimental.pallas import tpu_sc as plsc

NUM_CARDS = 1000000
HIDDEN_DIM = 32
BATCH = 4096
HIST = 200

_info = plsc.get_sparse_core_info()
NC, NS, L = _info.num_cores, _info.num_subcores, _info.num_lanes
NW = NC * NS

NB = BATCH // NW
B_PER_W = NB * HIST
TCH = 4
CROWS = TCH * NB
NCH = HIST // TCH
NPAIR = NCH // 2
DG = HIDDEN_DIM // 8


def _body(cards_hbm, emb_hbm, out_hbm, idx_all, idxT, rows0, rows1,
          tb0, tb1, tb2, tb3, sg0, sg1, sw0, sw1, sw2, sw3):
    wid = lax.axis_index("s") * NC + lax.axis_index("c")
    base = wid * B_PER_W
    iota = lax.iota(jnp.int32, L)

    pltpu.sync_copy(cards_hbm.at[pl.ds(base, B_PER_W)], idx_all)

    def build(i, _):
        t = i // (NB // L)
        bg = i % (NB // L)
        src = (bg * L + iota) * HIST + t
        vals = plsc.load_gather(idx_all, [src]) + 1
        idxT[pl.ds(t * NB + bg * L, L)] = vals
        return 0

    lax.fori_loop(0, HIST * (NB // L), build, 0, unroll=8)

    def idx_slice(c):
        return idxT.at[pl.ds(pl.multiple_of(c * CROWS, CROWS), CROWS)]

    def start_gather(c, rows, sem):
        pltpu.async_copy(emb_hbm.at[idx_slice(c)], rows, sem)

    def wait_gather(c, rows, sem):
        pltpu.make_async_copy(emb_hbm.at[idx_slice(c)], rows, sem).wait()

    def start_write(t, tb, sem):
        for g in range(DG):
            pltpu.async_copy(
                tb.at[pl.ds(g * 1024, 1024)], out_hbm.at[t, g, wid], sem
            )

    def wait_write(t, tb, sem):
        for g in range(DG):
            pltpu.make_async_copy(
                tb.at[pl.ds(g * 1024, 1024)], out_hbm.at[t, g, wid], sem
            ).wait()

    tbufs = (tb0, tb1, tb2, tb3)
    swaps = (sw0, sw1, sw2, sw3)
    posbase = iota * NB

    def transpose_t(rows, tl, tb):
        rbase = tl * NB

        def row(j, _):
            r = rbase + j
            v0 = rows[r, pl.ds(0, L)]
            v1 = rows[r, pl.ds(L, L)]
            pos = posbase + j
            plsc.store_scatter(tb, [pos], v0)
            plsc.store_scatter(tb, [pos + L * NB], v1)
            return 0

        lax.fori_loop(0, NB, row, 0, unroll=8)

    def handle_chunk(c, rows, rows_nxt, sem, sem_nxt):
        @pl.when(c + 1 < NCH)
        def _():
            start_gather(c + 1, rows_nxt, sem_nxt)

        wait_gather(c, rows, sem)
        for tl in range(TCH):
            t = c * TCH + tl

            @pl.when(c >= 1)
            def _():
                wait_write(t - TCH, tbufs[tl], swaps[tl])

            transpose_t(rows, tl, tbufs[tl])
            start_write(t, tbufs[tl], swaps[tl])

    start_gather(0, rows0, sg0)

    def pair(g, _):
        c0 = 2 * g
        handle_chunk(c0, rows0, rows1, sg0, sg1)
        handle_chunk(c0 + 1, rows1, rows0, sg1, sg0)
        return 0

    lax.fori_loop(0, NPAIR, pair, 0)

    for tl in range(TCH):
        t = HIST - TCH + tl
        wait_write(t, tbufs[tl], (sw0, sw1, sw2, sw3)[tl])


@jax.jit
def _embed(cards_flat, emb):
    mesh = plsc.VectorSubcoreMesh(core_axis_name="c", subcore_axis_name="s")
    fn = pl.kernel(
        _body,
        out_type=jax.ShapeDtypeStruct((HIST, DG, NW, 8 * NB), jnp.float32),
        mesh=mesh,
        scratch_types=[
            pltpu.VMEM((B_PER_W,), jnp.int32),
            pltpu.VMEM((B_PER_W,), jnp.int32),
            pltpu.VMEM((CROWS, HIDDEN_DIM), jnp.float32),
            pltpu.VMEM((CROWS, HIDDEN_DIM), jnp.float32),
            pltpu.VMEM((HIDDEN_DIM * NB,), jnp.float32),
            pltpu.VMEM((HIDDEN_DIM * NB,), jnp.float32),
            pltpu.VMEM((HIDDEN_DIM * NB,), jnp.float32),
            pltpu.VMEM((HIDDEN_DIM * NB,), jnp.float32),
            pltpu.SemaphoreType.DMA,
            pltpu.SemaphoreType.DMA,
            pltpu.SemaphoreType.DMA,
            pltpu.SemaphoreType.DMA,
            pltpu.SemaphoreType.DMA,
            pltpu.SemaphoreType.DMA,
        ],
        compiler_params=pltpu.CompilerParams(
            use_tc_tiling_on_sc=False, needs_layout_passes=False
        ),
    )
    return fn(cards_flat, emb)


def kernel(cards, emb):
    cards_flat = cards.reshape(-1).astype(jnp.int32)
    out4 = _embed(cards_flat, emb)
    out5 = out4.reshape(HIST, DG, NW, 8, NB)
    return out5.transpose(2, 4, 0, 1, 3).reshape(BATCH, HIST, HIDDEN_DIM)
